# baseline (speedup 1.0000x reference)
import functools

import jax
import jax.numpy as jnp
import numpy as np
from jax import lax
from jax.experimental import pallas as pl
from jax.experimental.pallas import tpu as pltpu
from jax.experimental.pallas import tpu_sc as plsc

_N_FINE = 100000
_NQ = 4096
_NG = 8192
_NW = 32
_WPT = 65536
_PPT = _N_FINE // _NW
_PITER = 3136
_FW = 9384


def _sc_body(qr_h, qs_h, gtr_h, gts_h, ov_h, rp_h, sp_h, t_h, cnt_h,
             qr_v, qs_v, gr_v, gs_v, ov_v, rp_v, sp_v, t_v, map_v, acc_v,
             sem0, sem1, sem2, sem3, sem4, sem5, sem6, sem7):
    cid = lax.axis_index("c")
    sid = lax.axis_index("s")
    wid = sid * 2 + cid
    start = 9375 * wid
    a = pl.multiple_of(jnp.minimum(jnp.bitwise_and(start, -8), 300000 - _FW), 8)
    ph = start - a

    crp = pltpu.async_copy(rp_h.at[pl.ds(a, _FW)], rp_v, sem5)
    csp = pltpu.async_copy(sp_h.at[pl.ds(a, _FW)], sp_v, sem6)
    ct = pltpu.async_copy(t_h, t_v, sem7)
    cqr = pltpu.async_copy(qr_h, qr_v, sem0)
    cqs = pltpu.async_copy(qs_h, qs_v, sem1)
    cgr = pltpu.async_copy(gtr_h, gr_v, sem2)
    cgs = pltpu.async_copy(gts_h, gs_v, sem3)
    cov = pltpu.async_copy(ov_h, ov_v, sem4)

    zeros16 = jnp.zeros((16,), jnp.int32)
    one16 = jnp.full((16,), 1, jnp.int32)
    iota16 = lax.iota(jnp.int32, 16)
    iota3 = iota16 * 3

    def _codes(rv, sv, i):
        r = rv[pl.ds(i, 16)]
        s = sv[pl.ds(i, 16)]
        code = r * 2048 + s
        own = lax.shift_right_logical(code, 17) == wid
        local = jnp.bitwise_and(lax.shift_right_logical(code, 1), 65535)
        return code, own, local

    cqr.wait()
    cqs.wait()

    @plsc.parallel_loop(0, _NQ, step=16, unroll=8)
    def _zero_q(i):
        _, own, local = _codes(qr_v, qs_v, i)
        plsc.store_scatter(map_v, [local], zeros16, mask=own)

    cgr.wait()
    cgs.wait()
    cov.wait()

    @plsc.parallel_loop(0, _NG, step=16, unroll=8)
    def _add_gt(i):
        code, own, local = _codes(gr_v, gs_v, i)
        o = ov_v[pl.ds(i, 16)]
        mask = jnp.logical_and(own, o > 0.0)
        val = lax.shift_left(one16, lax.shift_left(jnp.bitwise_and(code, 1), 4))
        plsc.addupdate_scatter(map_v, [local], val, mask=mask)

    @plsc.parallel_loop(0, _NQ, step=16, unroll=8,
                        carry=jnp.zeros((16,), jnp.float32))
    def _gather_q(i, acc):
        code, own, local = _codes(qr_v, qs_v, i)
        v = plsc.load_gather(map_v, [local], mask=own)
        half = jnp.bitwise_and(
            lax.shift_right_logical(v, lax.shift_left(jnp.bitwise_and(code, 1), 4)),
            65535)
        hit = jnp.logical_and(own, half > 0)
        return acc + jnp.where(hit, 1.0, 0.0).astype(jnp.float32)

    acc_v[...] = _gather_q
    pltpu.sync_copy(acc_v, cnt_h.at[pl.ds(wid * 16, 16)])

    crp.wait()
    csp.wait()
    ct.wait()
    tv = t_v[...]
    r00 = tv[0]
    r01 = tv[1]
    r02 = tv[2]
    t0 = tv[3]
    r10 = tv[4]
    r11 = tv[5]
    r12 = tv[6]
    t1 = tv[7]
    r20 = tv[8]
    r21 = tv[9]
    r22 = tv[10]
    t2 = tv[11]

    @plsc.parallel_loop(0, _PITER, step=16, unroll=4,
                        carry=jnp.zeros((16,), jnp.float32))
    def _fine(p, facc):
        valid = (p + iota16) < _PPT
        base = ph + 3 * p + iota3
        ix = jnp.where(valid, base, 0)
        xr = plsc.load_gather(rp_v, [ix], mask=valid)
        yr = plsc.load_gather(rp_v, [ix + 1], mask=valid)
        zr = plsc.load_gather(rp_v, [ix + 2], mask=valid)
        xs = plsc.load_gather(sp_v, [ix], mask=valid)
        ys = plsc.load_gather(sp_v, [ix + 1], mask=valid)
        zs = plsc.load_gather(sp_v, [ix + 2], mask=valid)
        dx = xr - (r00 * xs + r01 * ys + r02 * zs + t0)
        dy = yr - (r10 * xs + r11 * ys + r12 * zs + t1)
        dz = zr - (r20 * xs + r21 * ys + r22 * zs + t2)
        d2 = dx * dx + dy * dy + dz * dz
        hit = jnp.logical_and(valid, d2 < 0.01)
        return facc + jnp.where(hit, 1.0, 0.0).astype(jnp.float32)

    acc_v[...] = _fine
    pltpu.sync_copy(acc_v, cnt_h.at[pl.ds(_NW * 16 + wid * 16, 16)])


_sc_coarse = functools.partial(
    pl.kernel,
    out_type=jax.ShapeDtypeStruct((_NW * 16 * 2,), jnp.float32),
    mesh=plsc.VectorSubcoreMesh(core_axis_name="c", subcore_axis_name="s"),
    compiler_params=pltpu.CompilerParams(needs_layout_passes=False),
    scratch_types=[
        pltpu.VMEM((_NQ,), jnp.int32),
        pltpu.VMEM((_NQ,), jnp.int32),
        pltpu.VMEM((_NG,), jnp.int32),
        pltpu.VMEM((_NG,), jnp.int32),
        pltpu.VMEM((_NG,), jnp.float32),
        pltpu.VMEM((_FW,), jnp.float32),
        pltpu.VMEM((_FW,), jnp.float32),
        pltpu.VMEM((16,), jnp.float32),
        pltpu.VMEM((_WPT,), jnp.int32),
        pltpu.VMEM((16,), jnp.float32),
        pltpu.SemaphoreType.DMA,
        pltpu.SemaphoreType.DMA,
        pltpu.SemaphoreType.DMA,
        pltpu.SemaphoreType.DMA,
        pltpu.SemaphoreType.DMA,
        pltpu.SemaphoreType.DMA,
        pltpu.SemaphoreType.DMA,
        pltpu.SemaphoreType.DMA,
    ],
)(_sc_body)


def _euler_atan2_args(t_ref, et_ref, acc_num, acc_den):
    lane = jax.lax.broadcasted_iota(jnp.int32, (1, 128), 1)
    for base, ref in ((0, t_ref), (3, et_ref)):
        r00 = ref[0, 0]
        r10 = ref[1, 0]
        r20 = ref[2, 0]
        r21 = ref[2, 1]
        r22 = ref[2, 2]
        sy = jnp.sqrt(r00 * r00 + r10 * r10)
        acc_num = jnp.where(lane == base + 0, r21, acc_num)
        acc_den = jnp.where(lane == base + 0, r22, acc_den)
        acc_num = jnp.where(lane == base + 1, -r20, acc_num)
        acc_den = jnp.where(lane == base + 1, sy, acc_den)
        acc_num = jnp.where(lane == base + 2, r10, acc_num)
        acc_den = jnp.where(lane == base + 2, r00, acc_den)
    return acc_num, acc_den


def _tc_body(cc_ref, t_ref, et_ref, out_ref):
    c_precision = jnp.sum(cc_ref[0:4, :]) * (1.0 / _NQ)
    f_precision = jnp.sum(cc_ref[4:8, :]) * (1.0 / _N_FINE)

    num, den = _euler_atan2_args(t_ref, et_ref,
                                 jnp.zeros((1, 128), jnp.float32),
                                 jnp.ones((1, 128), jnp.float32))
    e = jnp.arctan2(num, den) * np.float32(180.0 / np.pi)
    lane = jax.lax.broadcasted_iota(jnp.int32, (1, 128), 1)
    e_est_shift = jnp.where(lane < 3, jnp.roll(e, -3, axis=1), 0.0)
    d_e = jnp.where(lane < 3, e - e_est_shift, 0.0)
    r_mse = jnp.sum(d_e * d_e) * (1.0 / 3.0)
    r_mae = jnp.sum(jnp.abs(d_e)) * (1.0 / 3.0)
    dt0 = t_ref[0, 3] - et_ref[0, 3]
    dt1 = t_ref[1, 3] - et_ref[1, 3]
    dt2 = t_ref[2, 3] - et_ref[2, 3]
    t_mse = (dt0 * dt0 + dt1 * dt1 + dt2 * dt2) * (1.0 / 3.0)
    t_mae = (jnp.abs(dt0) + jnp.abs(dt1) + jnp.abs(dt2)) * (1.0 / 3.0)

    out = jnp.zeros((1, 128), jnp.float32)
    out = jnp.where(lane == 0, c_precision, out)
    out = jnp.where(lane == 1, f_precision, out)
    out = jnp.where(lane == 2, r_mse, out)
    out = jnp.where(lane == 3, r_mae, out)
    out = jnp.where(lane == 4, t_mse, out)
    out = jnp.where(lane == 5, t_mae, out)
    out_ref[...] = out


def kernel(ref_points_c, src_points_c, gt_node_corr_overlaps,
           gt_node_corr_indices, ref_node_corr_indices, src_node_corr_indices,
           ref_corr_points, src_corr_points, transform, estimated_transform):
    del ref_points_c, src_points_c

    cnt = _sc_coarse(
        ref_node_corr_indices.astype(jnp.int32),
        src_node_corr_indices.astype(jnp.int32),
        gt_node_corr_indices[:, 0].astype(jnp.int32),
        gt_node_corr_indices[:, 1].astype(jnp.int32),
        gt_node_corr_overlaps,
        ref_corr_points.reshape(3 * _N_FINE),
        src_corr_points.reshape(3 * _N_FINE),
        transform.reshape(16),
    )
    cc = cnt.reshape(8, 128)

    smem_spec = pl.BlockSpec(memory_space=pltpu.SMEM)
    vmem_spec = pl.BlockSpec(memory_space=pltpu.VMEM)
    out = pl.pallas_call(
        _tc_body,
        out_shape=jax.ShapeDtypeStruct((1, 128), jnp.float32),
        in_specs=[vmem_spec, smem_spec, smem_spec],
        out_specs=vmem_spec,
    )(cc, transform, estimated_transform)
    return out[0, 0:6]


# speedup vs baseline: 4.8426x; 4.8426x over previous
import functools

import jax
import jax.numpy as jnp
import numpy as np
from jax import lax
from jax.experimental import pallas as pl
from jax.experimental.pallas import tpu as pltpu
from jax.experimental.pallas import tpu_sc as plsc

_N_FINE = 100000
_NQ = 4096
_NG = 8192
_NW = 32
_WPT = 65536


def _sc_body(qr_h, qs_h, gtr_h, gts_h, ov_h, cnt_h,
             qr_v, qs_v, gr_v, gs_v, ov_v, map_v, acc_v,
             sem0, sem1, sem2, sem3, sem4):
    cid = lax.axis_index("c")
    sid = lax.axis_index("s")
    wid = sid * 2 + cid

    cqr = pltpu.async_copy(qr_h, qr_v, sem0)
    cqs = pltpu.async_copy(qs_h, qs_v, sem1)
    cgr = pltpu.async_copy(gtr_h, gr_v, sem2)
    cgs = pltpu.async_copy(gts_h, gs_v, sem3)
    cov = pltpu.async_copy(ov_h, ov_v, sem4)

    zeros16 = jnp.zeros((16,), jnp.int32)
    one16 = jnp.full((16,), 1, jnp.int32)

    def _codes(rv, sv, i):
        r = rv[pl.ds(i, 16)]
        s = sv[pl.ds(i, 16)]
        code = r * 2048 + s
        own = lax.shift_right_logical(code, 17) == wid
        local = jnp.bitwise_and(lax.shift_right_logical(code, 1), 65535)
        return code, own, local

    cqr.wait()
    cqs.wait()

    @plsc.parallel_loop(0, _NQ, step=16, unroll=8)
    def _zero_q(i):
        _, own, local = _codes(qr_v, qs_v, i)
        plsc.store_scatter(map_v, [local], zeros16, mask=own)

    cgr.wait()
    cgs.wait()
    cov.wait()

    @plsc.parallel_loop(0, _NG, step=16, unroll=8)
    def _add_gt(i):
        code, own, local = _codes(gr_v, gs_v, i)
        o = ov_v[pl.ds(i, 16)]
        mask = jnp.logical_and(own, o > 0.0)
        val = lax.shift_left(one16, lax.shift_left(jnp.bitwise_and(code, 1), 4))
        plsc.addupdate_scatter(map_v, [local], val, mask=mask)

    @plsc.parallel_loop(0, _NQ, step=16, unroll=8,
                        carry=jnp.zeros((16,), jnp.float32))
    def _gather_q(i, acc):
        code, own, local = _codes(qr_v, qs_v, i)
        v = plsc.load_gather(map_v, [local], mask=own)
        half = jnp.bitwise_and(
            lax.shift_right_logical(v, lax.shift_left(jnp.bitwise_and(code, 1), 4)),
            65535)
        hit = jnp.logical_and(own, half > 0)
        return acc + jnp.where(hit, 1.0, 0.0).astype(jnp.float32)

    acc_v[...] = _gather_q
    pltpu.sync_copy(acc_v, cnt_h.at[pl.ds(wid * 16, 16)])


_sc_coarse = functools.partial(
    pl.kernel,
    out_type=jax.ShapeDtypeStruct((_NW * 16,), jnp.float32),
    mesh=plsc.VectorSubcoreMesh(core_axis_name="c", subcore_axis_name="s"),
    compiler_params=pltpu.CompilerParams(needs_layout_passes=False),
    scratch_types=[
        pltpu.VMEM((_NQ,), jnp.int32),
        pltpu.VMEM((_NQ,), jnp.int32),
        pltpu.VMEM((_NG,), jnp.int32),
        pltpu.VMEM((_NG,), jnp.int32),
        pltpu.VMEM((_NG,), jnp.float32),
        pltpu.VMEM((_WPT,), jnp.int32),
        pltpu.VMEM((16,), jnp.float32),
        pltpu.SemaphoreType.DMA,
        pltpu.SemaphoreType.DMA,
        pltpu.SemaphoreType.DMA,
        pltpu.SemaphoreType.DMA,
        pltpu.SemaphoreType.DMA,
    ],
)(_sc_body)


def _euler_atan2_args(t_ref, et_ref, acc_num, acc_den):
    lane = jax.lax.broadcasted_iota(jnp.int32, (1, 128), 1)
    for base, ref in ((0, t_ref), (3, et_ref)):
        r00 = ref[0, 0]
        r10 = ref[1, 0]
        r20 = ref[2, 0]
        r21 = ref[2, 1]
        r22 = ref[2, 2]
        sy = jnp.sqrt(r00 * r00 + r10 * r10)
        acc_num = jnp.where(lane == base + 0, r21, acc_num)
        acc_den = jnp.where(lane == base + 0, r22, acc_den)
        acc_num = jnp.where(lane == base + 1, -r20, acc_num)
        acc_den = jnp.where(lane == base + 1, sy, acc_den)
        acc_num = jnp.where(lane == base + 2, r10, acc_num)
        acc_den = jnp.where(lane == base + 2, r00, acc_den)
    return acc_num, acc_den


def _tc_body(ref_f, src_f, cc_ref, t_ref, et_ref, out_ref):
    rx = ref_f[0:1, :]
    ry = ref_f[1:2, :]
    rz = ref_f[2:3, :]
    sx = src_f[0:1, :]
    sy_ = src_f[1:2, :]
    sz = src_f[2:3, :]
    dx = rx - (t_ref[0, 0] * sx + t_ref[0, 1] * sy_ + t_ref[0, 2] * sz + t_ref[0, 3])
    dy = ry - (t_ref[1, 0] * sx + t_ref[1, 1] * sy_ + t_ref[1, 2] * sz + t_ref[1, 3])
    dz = rz - (t_ref[2, 0] * sx + t_ref[2, 1] * sy_ + t_ref[2, 2] * sz + t_ref[2, 3])
    d2 = dx * dx + dy * dy + dz * dz
    f_count = jnp.sum(jnp.where(d2 < 0.01, 1.0, 0.0))
    f_precision = f_count * (1.0 / _N_FINE)

    c_precision = jnp.sum(cc_ref[...]) * (1.0 / _NQ)

    num, den = _euler_atan2_args(t_ref, et_ref,
                                 jnp.zeros((1, 128), jnp.float32),
                                 jnp.ones((1, 128), jnp.float32))
    e = jnp.arctan2(num, den) * np.float32(180.0 / np.pi)
    lane = jax.lax.broadcasted_iota(jnp.int32, (1, 128), 1)
    e_est_shift = jnp.where(lane < 3, jnp.roll(e, -3, axis=1), 0.0)
    d_e = jnp.where(lane < 3, e - e_est_shift, 0.0)
    r_mse = jnp.sum(d_e * d_e) * (1.0 / 3.0)
    r_mae = jnp.sum(jnp.abs(d_e)) * (1.0 / 3.0)
    dt0 = t_ref[0, 3] - et_ref[0, 3]
    dt1 = t_ref[1, 3] - et_ref[1, 3]
    dt2 = t_ref[2, 3] - et_ref[2, 3]
    t_mse = (dt0 * dt0 + dt1 * dt1 + dt2 * dt2) * (1.0 / 3.0)
    t_mae = (jnp.abs(dt0) + jnp.abs(dt1) + jnp.abs(dt2)) * (1.0 / 3.0)

    out = jnp.zeros((1, 128), jnp.float32)
    out = jnp.where(lane == 0, c_precision, out)
    out = jnp.where(lane == 1, f_precision, out)
    out = jnp.where(lane == 2, r_mse, out)
    out = jnp.where(lane == 3, r_mae, out)
    out = jnp.where(lane == 4, t_mse, out)
    out = jnp.where(lane == 5, t_mae, out)
    out_ref[...] = out


def kernel(ref_points_c, src_points_c, gt_node_corr_overlaps,
           gt_node_corr_indices, ref_node_corr_indices, src_node_corr_indices,
           ref_corr_points, src_corr_points, transform, estimated_transform):
    del ref_points_c, src_points_c

    cnt = _sc_coarse(
        ref_node_corr_indices.astype(jnp.int32),
        src_node_corr_indices.astype(jnp.int32),
        gt_node_corr_indices[:, 0].astype(jnp.int32),
        gt_node_corr_indices[:, 1].astype(jnp.int32),
        gt_node_corr_overlaps,
    )
    cc = cnt.reshape(4, 128)
    ref_f = ref_corr_points.T
    src_f = src_corr_points.T

    smem_spec = pl.BlockSpec(memory_space=pltpu.SMEM)
    vmem_spec = pl.BlockSpec(memory_space=pltpu.VMEM)
    out = pl.pallas_call(
        _tc_body,
        out_shape=jax.ShapeDtypeStruct((1, 128), jnp.float32),
        in_specs=[vmem_spec, vmem_spec, vmem_spec, smem_spec, smem_spec],
        out_specs=vmem_spec,
    )(ref_f, src_f, cc, transform, estimated_transform)
    return out[0, 0:6]
